# baseline (speedup 1.0000x reference)
import functools

import jax
import jax.numpy as jnp
from jax import lax
from jax.experimental import pallas as pl
from jax.experimental.pallas import tpu as pltpu
from jax.experimental.pallas import tpu_sc as plsc

_NUM_CORES = 2
_NUM_SUBCORES = 16
_NUM_WORKERS = _NUM_CORES * _NUM_SUBCORES
_LANES = 16
_CHUNK = 128
_NB = 8
_AHEAD = 4


@functools.partial(jax.jit, static_argnums=(2, 3))
def _sc_gather(idx, weight, n_chunks, d):
    mesh = plsc.VectorSubcoreMesh(core_axis_name="c", subcore_axis_name="s")

    @functools.partial(
        pl.kernel,
        mesh=mesh,
        out_type=jax.ShapeDtypeStruct(
            (_NUM_WORKERS * n_chunks, _CHUNK, d), jnp.float32
        ),
        scratch_types=[
            pltpu.VMEM((n_chunks, _CHUNK), jnp.int32),
            pltpu.VMEM((_NB, _CHUNK, d), jnp.float32),
        ] + [pltpu.SemaphoreType.DMA] * (2 * _NB),
        compiler_params=pltpu.CompilerParams(use_tc_tiling_on_sc=False),
    )
    def k(idx_hbm, table_hbm, out_hbm, idx_v, rows_v, *sems):
        gsems = sems[:_NB]
        wsems = sems[_NB:]
        wid = lax.axis_index("s") * _NUM_CORES + lax.axis_index("c")
        base = wid * n_chunks
        pltpu.sync_copy(idx_hbm.at[wid], idx_v)

        def fire_gather(j, b):
            for i in range(_CHUNK // _LANES):
                idx16 = idx_v[j, pl.ds(i * _LANES, _LANES)]
                pltpu.async_copy(
                    table_hbm.at[idx16],
                    rows_v.at[b, pl.ds(i * _LANES, _LANES)],
                    gsems[b],
                )

        def wait_gather(b):
            pltpu.make_async_copy(
                out_hbm.at[base], rows_v.at[b], gsems[b]
            ).wait()

        for b in range(_NB):
            fire_gather(b, b)

        def group(g, carry):
            j0 = g * _NB
            for b in range(_NB):
                j = j0 + b
                wait_gather(b)
                pltpu.async_copy(rows_v.at[b], out_hbm.at[base + j], wsems[b])
                jmid = j - _AHEAD
                bmid = (b - _AHEAD) % _NB

                @pl.when(jnp.logical_and(jmid >= 0, jmid + _NB < n_chunks))
                def _():
                    pltpu.make_async_copy(
                        rows_v.at[bmid], out_hbm.at[base], wsems[bmid]
                    ).wait()
                    fire_gather(jmid + _NB, bmid)

            return carry

        lax.fori_loop(0, n_chunks // _NB, group, 0)

        for b in range(_NB):
            pltpu.make_async_copy(
                rows_v.at[b], out_hbm.at[base], wsems[b]
            ).wait()

    return k(idx, weight)


def kernel(x, weight):
    b, f = x.shape
    v, d = weight.shape
    bf = b * f
    assert bf % (_NUM_WORKERS * _CHUNK * _NB) == 0
    n_chunks = bf // (_NUM_WORKERS * _CHUNK)
    idx = x.reshape(_NUM_WORKERS, n_chunks, _CHUNK).astype(jnp.int32)
    out = _sc_gather(idx, weight, n_chunks, d)
    return out.reshape(b, f, d)
